# baseline (speedup 1.0000x reference)
import functools

import jax
import jax.numpy as jnp
from jax import lax
from jax.experimental import pallas as pl
from jax.experimental.pallas import tpu as pltpu
from jax.experimental.pallas import tpu_sc as plsc

BATCH = 16384
HIST = 50
EMB_DIM = 31
OUT_DIM = 32
TOTAL = BATCH * HIST
NUM_WORKERS = 32
CHUNK = 128
CHUNKS_PER_WORKER = TOTAL // (NUM_WORKERS * CHUNK)
LANES = 16


def _sc_embed_body(idx_hbm, table_hbm, out_hbm, idx_v, rows_v, out_v, sem):
    wid = lax.axis_index("s") * 2 + lax.axis_index("c")
    chunk_base = wid * CHUNKS_PER_WORKER

    pltpu.sync_copy(idx_hbm.at[pl.ds(chunk_base, CHUNKS_PER_WORKER)], idx_v)

    lane = lax.iota(jnp.int32, LANES)

    @pl.loop(0, CHUNKS_PER_WORKER)
    def _chunk(j):
        pltpu.async_copy(table_hbm.at[idx_v.at[j]], rows_v, sem).wait()

        @pl.loop(0, CHUNK // LANES)
        def _group(g):
            rid = lane + (g * LANES)
            es = []
            nsq = jnp.zeros((LANES,), jnp.float32)
            for d in range(EMB_DIM):
                col = jnp.full((LANES,), d, jnp.int32)
                e = plsc.load_gather(rows_v, [rid, col])
                es.append(e)
                nsq = nsq + e * e
            inv = 1.0 / (nsq + 1.0)
            two_inv = inv + inv
            for d in range(EMB_DIM):
                col = jnp.full((LANES,), d, jnp.int32)
                plsc.store_scatter(out_v, [rid, col], es[d] * two_inv)
            last = jnp.full((LANES,), EMB_DIM, jnp.int32)
            plsc.store_scatter(out_v, [rid, last], (nsq - 1.0) * inv)
            return None

        row0 = (chunk_base + j) * CHUNK
        pltpu.sync_copy(out_v, out_hbm.at[pl.ds(row0, CHUNK)])


@jax.jit
def _sc_embed(idx, table32):
    mesh = plsc.VectorSubcoreMesh(core_axis_name="c", subcore_axis_name="s")
    f = functools.partial(
        pl.kernel,
        out_type=jax.ShapeDtypeStruct((TOTAL, OUT_DIM), jnp.float32),
        mesh=mesh,
        scratch_types=[
            pltpu.VMEM((CHUNKS_PER_WORKER, CHUNK), jnp.int32),
            pltpu.VMEM((CHUNK, OUT_DIM), jnp.float32),
            pltpu.VMEM((CHUNK, OUT_DIM), jnp.float32),
            pltpu.SemaphoreType.DMA,
        ],
        compiler_params=pltpu.CompilerParams(
            needs_layout_passes=False, use_tc_tiling_on_sc=False
        ),
    )(_sc_embed_body)
    return f(idx, table32)


def kernel(inputs, table):
    idx = inputs.astype(jnp.int32).reshape(TOTAL // CHUNK, CHUNK)
    table32 = jnp.pad(table, ((0, 0), (0, 1)))
    out = _sc_embed(idx, table32)
    return out.reshape(BATCH, HIST, OUT_DIM)


# speedup vs baseline: 1.4272x; 1.4272x over previous
import functools

import jax
import jax.numpy as jnp
from jax import lax
from jax.experimental import pallas as pl
from jax.experimental.pallas import tpu as pltpu
from jax.experimental.pallas import tpu_sc as plsc

BATCH = 16384
HIST = 50
EMB_DIM = 31
OUT_DIM = 32
NUM_WORKERS = 32
IDX_MINOR = 100
IDX_ROWS = BATCH * HIST // IDX_MINOR
ROWS_PER_WORKER = BATCH * HIST // NUM_WORKERS
BPC = 8
CHUNK = BPC * HIST
NCHUNK = ROWS_PER_WORKER // CHUNK
GPC = CHUNK // 16
LANES = 16


def _sc_embed_body(idx_hbm, table_hbm, out_hbm,
                   idx_v, rows_a, rows_b, out_a, out_b,
                   gsem_a, gsem_b, wsem_a, wsem_b):
    wid = lax.axis_index("s") * 2 + lax.axis_index("c")
    idx_row0 = wid * (NCHUNK * CHUNK // IDX_MINOR)
    batch0 = wid * (NCHUNK * BPC)

    pltpu.sync_copy(
        idx_hbm.at[pl.ds(idx_row0, NCHUNK * CHUNK // IDX_MINOR)], idx_v)

    lane = lax.iota(jnp.int32, LANES)
    bufs = ((rows_a, out_a, gsem_a, wsem_a), (rows_b, out_b, gsem_b, wsem_b))

    def start_gather(ci, rows, gsem):
        for k in range(CHUNK // IDX_MINOR):
            pltpu.async_copy(
                table_hbm.at[idx_v.at[ci * (CHUNK // IDX_MINOR) + k]],
                rows.at[pl.ds(k * IDX_MINOR, IDX_MINOR)], gsem)

    def drain_gather(ci, rows, gsem):
        for k in range(CHUNK // IDX_MINOR):
            pltpu.make_async_copy(
                table_hbm.at[idx_v.at[ci * (CHUNK // IDX_MINOR) + k]],
                rows.at[pl.ds(k * IDX_MINOR, IDX_MINOR)], gsem).wait()

    def compute(rows, out):
        @pl.loop(0, GPC)
        def _grp(g):
            rid = lane + g * LANES
            rb = rid // HIST
            rl = rid - rb * HIST
            es = []
            nsq = jnp.zeros((LANES,), jnp.float32)
            for d in range(EMB_DIM):
                col = jnp.full((LANES,), d, jnp.int32)
                e = plsc.load_gather(rows, [rid, col])
                es.append(e)
                nsq = nsq + e * e
            inv = 1.0 / (nsq + 1.0)
            two_inv = inv + inv
            for d in range(EMB_DIM):
                col = jnp.full((LANES,), d, jnp.int32)
                plsc.store_scatter(out, [rb, rl, col], es[d] * two_inv)
            last = jnp.full((LANES,), EMB_DIM, jnp.int32)
            plsc.store_scatter(out, [rb, rl, last], (nsq - 1.0) * inv)

    start_gather(0, rows_a, gsem_a)
    start_gather(1, rows_b, gsem_b)

    @pl.loop(0, NCHUNK // 2)
    def _pair(p2):
        for p in (0, 1):
            rows, out, gsem, wsem = bufs[p]
            ci = p2 * 2 + p
            drain_gather(ci, rows, gsem)

            @pl.when(ci >= 2)
            def _():
                pltpu.make_async_copy(
                    out,
                    out_hbm.at[pl.ds(batch0 + (ci - 2) * BPC, BPC)],
                    wsem).wait()

            compute(rows, out)
            pltpu.async_copy(
                out, out_hbm.at[pl.ds(batch0 + ci * BPC, BPC)], wsem)

            @pl.when(ci + 2 < NCHUNK)
            def _():
                start_gather(ci + 2, rows, gsem)

    pltpu.make_async_copy(
        out_a, out_hbm.at[pl.ds(batch0 + (NCHUNK - 2) * BPC, BPC)],
        wsem_a).wait()
    pltpu.make_async_copy(
        out_b, out_hbm.at[pl.ds(batch0 + (NCHUNK - 1) * BPC, BPC)],
        wsem_b).wait()


@jax.jit
def _sc_embed(idx, table32):
    mesh = plsc.VectorSubcoreMesh(core_axis_name="c", subcore_axis_name="s")
    f = functools.partial(
        pl.kernel,
        out_type=jax.ShapeDtypeStruct((BATCH, HIST, OUT_DIM), jnp.float32),
        mesh=mesh,
        scratch_types=[
            pltpu.VMEM((ROWS_PER_WORKER // IDX_MINOR, IDX_MINOR), jnp.int32),
            pltpu.VMEM((CHUNK, OUT_DIM), jnp.float32),
            pltpu.VMEM((CHUNK, OUT_DIM), jnp.float32),
            pltpu.VMEM((BPC, HIST, OUT_DIM), jnp.float32),
            pltpu.VMEM((BPC, HIST, OUT_DIM), jnp.float32),
            pltpu.SemaphoreType.DMA,
            pltpu.SemaphoreType.DMA,
            pltpu.SemaphoreType.DMA,
            pltpu.SemaphoreType.DMA,
        ],
        compiler_params=pltpu.CompilerParams(
            needs_layout_passes=False, use_tc_tiling_on_sc=False
        ),
    )(_sc_embed_body)
    return f(idx, table32)


def kernel(inputs, table):
    idx = inputs.astype(jnp.int32).reshape(IDX_ROWS, IDX_MINOR)
    table32 = jnp.pad(table, ((0, 0), (0, 1)))
    return _sc_embed(idx, table32)


# speedup vs baseline: 1.4334x; 1.0044x over previous
import functools

import jax
import jax.numpy as jnp
from jax import lax
from jax.experimental import pallas as pl
from jax.experimental.pallas import tpu as pltpu
from jax.experimental.pallas import tpu_sc as plsc

BATCH = 16384
HIST = 50
EMB_DIM = 31
OUT_DIM = 32
TOTAL = BATCH * HIST
NUM_WORKERS = 32
ROWS_PER_WORKER = TOTAL // NUM_WORKERS
CHUNK = 512
GSUB = 128
NCHUNK = ROWS_PER_WORKER // CHUNK
GPC = CHUNK // 16
LANES = 16


def _sc_embed_body(idx_hbm, table_hbm, out_hbm,
                   idx_v, rows_a, rows_b, out_a, out_b,
                   gsem_a, gsem_b, wsem_a, wsem_b):
    wid = lax.axis_index("s") * 2 + lax.axis_index("c")
    row0 = wid * ROWS_PER_WORKER

    pltpu.sync_copy(idx_hbm.at[pl.ds(row0, ROWS_PER_WORKER)], idx_v)

    lane = lax.iota(jnp.int32, LANES)
    bufs = ((rows_a, out_a, gsem_a, wsem_a), (rows_b, out_b, gsem_b, wsem_b))

    def start_gather(ci, rows, gsem):
        for k in range(CHUNK // GSUB):
            pltpu.async_copy(
                table_hbm.at[idx_v.at[pl.ds(ci * CHUNK + k * GSUB, GSUB)]],
                rows.at[pl.ds(k * GSUB, GSUB)], gsem)

    def drain_gather(ci, rows, gsem):
        for k in range(CHUNK // GSUB):
            pltpu.make_async_copy(
                table_hbm.at[idx_v.at[pl.ds(ci * CHUNK + k * GSUB, GSUB)]],
                rows.at[pl.ds(k * GSUB, GSUB)], gsem).wait()

    def compute(rows, out):
        @pl.loop(0, GPC)
        def _grp(g):
            rid = lane + g * LANES
            fl = rid * OUT_DIM
            es = []
            nsq = jnp.zeros((LANES,), jnp.float32)
            for d in range(EMB_DIM):
                col = jnp.full((LANES,), d, jnp.int32)
                e = plsc.load_gather(rows, [rid, col])
                es.append(e)
                nsq = nsq + e * e
            inv = 1.0 / (nsq + 1.0)
            two_inv = inv + inv
            for d in range(EMB_DIM):
                plsc.store_scatter(out, [fl + d], es[d] * two_inv)
            plsc.store_scatter(out, [fl + EMB_DIM], (nsq - 1.0) * inv)

    start_gather(0, rows_a, gsem_a)
    start_gather(1, rows_b, gsem_b)

    @pl.loop(0, NCHUNK // 2)
    def _pair(p2):
        for p in (0, 1):
            rows, out, gsem, wsem = bufs[p]
            ci = p2 * 2 + p
            drain_gather(ci, rows, gsem)

            @pl.when(ci >= 2)
            def _():
                pltpu.make_async_copy(
                    out,
                    out_hbm.at[pl.ds((row0 + (ci - 2) * CHUNK) * OUT_DIM,
                                     CHUNK * OUT_DIM)],
                    wsem).wait()

            compute(rows, out)
            pltpu.async_copy(
                out,
                out_hbm.at[pl.ds((row0 + ci * CHUNK) * OUT_DIM,
                                 CHUNK * OUT_DIM)],
                wsem)

            @pl.when(ci + 2 < NCHUNK)
            def _():
                start_gather(ci + 2, rows, gsem)

    pltpu.make_async_copy(
        out_a,
        out_hbm.at[pl.ds((row0 + (NCHUNK - 2) * CHUNK) * OUT_DIM,
                         CHUNK * OUT_DIM)],
        wsem_a).wait()
    pltpu.make_async_copy(
        out_b,
        out_hbm.at[pl.ds((row0 + (NCHUNK - 1) * CHUNK) * OUT_DIM,
                         CHUNK * OUT_DIM)],
        wsem_b).wait()


@jax.jit
def _sc_embed(idx, table32):
    mesh = plsc.VectorSubcoreMesh(core_axis_name="c", subcore_axis_name="s")
    f = functools.partial(
        pl.kernel,
        out_type=jax.ShapeDtypeStruct((TOTAL * OUT_DIM,), jnp.float32),
        mesh=mesh,
        scratch_types=[
            pltpu.VMEM((ROWS_PER_WORKER,), jnp.int32),
            pltpu.VMEM((CHUNK, OUT_DIM), jnp.float32),
            pltpu.VMEM((CHUNK, OUT_DIM), jnp.float32),
            pltpu.VMEM((CHUNK * OUT_DIM,), jnp.float32),
            pltpu.VMEM((CHUNK * OUT_DIM,), jnp.float32),
            pltpu.SemaphoreType.DMA,
            pltpu.SemaphoreType.DMA,
            pltpu.SemaphoreType.DMA,
            pltpu.SemaphoreType.DMA,
        ],
        compiler_params=pltpu.CompilerParams(
            needs_layout_passes=False, use_tc_tiling_on_sc=False
        ),
    )(_sc_embed_body)
    return f(idx, table32)


def kernel(inputs, table):
    idx = inputs.astype(jnp.int32).reshape(TOTAL)
    table32 = jnp.pad(table, ((0, 0), (0, 1)))
    out = _sc_embed(idx, table32)
    return out.reshape(BATCH, HIST, OUT_DIM)


# speedup vs baseline: 1.9841x; 1.3842x over previous
import functools

import jax
import jax.numpy as jnp
from jax import lax
from jax.experimental import pallas as pl
from jax.experimental.pallas import tpu as pltpu
from jax.experimental.pallas import tpu_sc as plsc

BATCH = 16384
HIST = 50
EMB_DIM = 31
OUT_DIM = 32
TOTAL = BATCH * HIST
NUM_WORKERS = 32
ROWS_PER_WORKER = TOTAL // NUM_WORKERS
CHUNK = 256
NCHUNK = ROWS_PER_WORKER // CHUNK
GPC = CHUNK // 16
LANES = 16


def _sc_embed_body(idx_hbm, table_hbm, out_hbm,
                   idx_v, rows_a, rows_b, out_a, out_b,
                   gsem_a, gsem_b, wsem_a, wsem_b):
    wid = lax.axis_index("s") * 2 + lax.axis_index("c")
    row0 = wid * ROWS_PER_WORKER

    pltpu.sync_copy(idx_hbm.at[pl.ds(row0, ROWS_PER_WORKER)], idx_v)

    lane = lax.iota(jnp.int32, LANES)
    bufs = ((rows_a, out_a, gsem_a, wsem_a), (rows_b, out_b, gsem_b, wsem_b))

    def start_gather(ci, rows, gsem):
        @pl.loop(0, CHUNK // LANES)
        def _g(g):
            vec = idx_v[pl.ds(ci * CHUNK + g * LANES, LANES)]
            for k in range(LANES):
                i = vec[k]
                t = lax.shift_right_logical(i, 3)
                s = lax.bitwise_and(i, 7)
                pltpu.async_copy(
                    table_hbm.at[pl.ds(t, 1), s],
                    rows.at[pl.ds(g * LANES + k, 1), :], gsem)

    def drain_gather(rows, gsem):
        pltpu.make_async_copy(
            table_hbm.at[pl.ds(0, CHUNK), 0], rows, gsem).wait()

    def compute(rows, out):
        @pl.loop(0, GPC)
        def _grp(g):
            rid = lane + g * LANES
            fl = rid * OUT_DIM
            es = []
            nsq = jnp.zeros((LANES,), jnp.float32)
            for d in range(EMB_DIM):
                col = jnp.full((LANES,), d, jnp.int32)
                e = plsc.load_gather(rows, [rid, col])
                es.append(e)
                nsq = nsq + e * e
            inv = 1.0 / (nsq + 1.0)
            two_inv = inv + inv
            for d in range(EMB_DIM):
                plsc.store_scatter(out, [fl + d], es[d] * two_inv)
            plsc.store_scatter(out, [fl + EMB_DIM], (nsq - 1.0) * inv)

    start_gather(0, rows_a, gsem_a)
    start_gather(1, rows_b, gsem_b)

    @pl.loop(0, NCHUNK // 2)
    def _pair(p2):
        for p in (0, 1):
            rows, out, gsem, wsem = bufs[p]
            ci = p2 * 2 + p
            drain_gather(rows, gsem)

            @pl.when(ci >= 2)
            def _():
                pltpu.make_async_copy(
                    out,
                    out_hbm.at[pl.ds((row0 + (ci - 2) * CHUNK) * OUT_DIM,
                                     CHUNK * OUT_DIM)],
                    wsem).wait()

            compute(rows, out)
            pltpu.async_copy(
                out,
                out_hbm.at[pl.ds((row0 + ci * CHUNK) * OUT_DIM,
                                 CHUNK * OUT_DIM)],
                wsem)

            @pl.when(ci + 2 < NCHUNK)
            def _():
                start_gather(ci + 2, rows, gsem)

    pltpu.make_async_copy(
        out_a,
        out_hbm.at[pl.ds((row0 + (NCHUNK - 2) * CHUNK) * OUT_DIM,
                         CHUNK * OUT_DIM)],
        wsem_a).wait()
    pltpu.make_async_copy(
        out_b,
        out_hbm.at[pl.ds((row0 + (NCHUNK - 1) * CHUNK) * OUT_DIM,
                         CHUNK * OUT_DIM)],
        wsem_b).wait()


@jax.jit
def _sc_embed(idx, table):
    mesh = plsc.VectorSubcoreMesh(core_axis_name="c", subcore_axis_name="s")
    f = functools.partial(
        pl.kernel,
        out_type=jax.ShapeDtypeStruct((TOTAL * OUT_DIM,), jnp.float32),
        mesh=mesh,
        scratch_types=[
            pltpu.VMEM((ROWS_PER_WORKER,), jnp.int32),
            pltpu.VMEM((CHUNK, EMB_DIM), jnp.float32),
            pltpu.VMEM((CHUNK, EMB_DIM), jnp.float32),
            pltpu.VMEM((CHUNK * OUT_DIM,), jnp.float32),
            pltpu.VMEM((CHUNK * OUT_DIM,), jnp.float32),
            pltpu.SemaphoreType.DMA,
            pltpu.SemaphoreType.DMA,
            pltpu.SemaphoreType.DMA,
            pltpu.SemaphoreType.DMA,
        ],
        compiler_params=pltpu.CompilerParams(
            needs_layout_passes=False, use_tc_tiling_on_sc=True
        ),
    )(_sc_embed_body)
    return f(idx, table)


def kernel(inputs, table):
    idx = inputs.astype(jnp.int32).reshape(TOTAL)
    table3 = table.reshape(125000, 8, EMB_DIM)
    out = _sc_embed(idx, table3)
    return out.reshape(BATCH, HIST, OUT_DIM)
